# baseline (speedup 1.0000x reference)
import functools

import jax
import jax.numpy as jnp
from jax import lax
from jax.experimental import pallas as pl
from jax.experimental.pallas import tpu as pltpu
from jax.experimental.pallas import tpu_sc as plsc

_NC = 2
_NS = 16
_L = 16
_R = 5
_D = 128
_CHK = 80


def _mesh():
    return plsc.VectorSubcoreMesh(core_axis_name="c", subcore_axis_name="s")


def _cnt_pass(et, dst, oh, zeros_nd):
    E = et.shape[0]
    N = zeros_nd.shape[0]
    NW = _NC * _NS
    epw = E // NW
    nit = epw // _CHK
    rpt = N // _NS
    crow = 128
    cpt = rpt // crow

    @functools.partial(
        pl.kernel,
        out_type=jax.ShapeDtypeStruct((_NC, N, _D), jnp.float32),
        mesh=_mesh(),
        scratch_types=[
            pltpu.VMEM_SHARED((N, _D), jnp.float32),
            pltpu.VMEM((_CHK,), jnp.int32),
            pltpu.VMEM((_CHK,), jnp.int32),
            pltpu.VMEM((_CHK, _D), jnp.float32),
            pltpu.VMEM((crow, _D), jnp.float32),
            pltpu.SemaphoreType.DMA,
        ],
    )
    def k(et_hbm, dst_hbm, oh_hbm, z_hbm, out_hbm, acc, tv, dv, ohrows, obuf, sem):
        c = lax.axis_index("c")
        s = lax.axis_index("s")
        wid = s * _NC + c
        rowbase = s * rpt
        pltpu.sync_copy(z_hbm.at[pl.ds(rowbase, rpt)], acc.at[pl.ds(rowbase, rpt)])
        plsc.subcore_barrier()
        ebase = wid * epw
        nrep = oh_hbm.shape[0] // _R

        def body(i, carry):
            off = ebase + i * _CHK
            pltpu.sync_copy(et_hbm.at[pl.ds(off, _CHK)], tv)
            pltpu.sync_copy(dst_hbm.at[pl.ds(off, _CHK)], dv)
            rot = (i * (_CHK // _L) + wid) * 7
            for j in range(_CHK // _L):
                sl = pl.ds(j * _L, _L)
                m = lax.rem(lax.iota(jnp.int32, _L) * 3 + rot + j, nrep)
                tv[sl] = tv[sl] + m * _R
            pltpu.async_copy(oh_hbm.at[tv], ohrows, sem).wait()
            pltpu.sync_copy(ohrows, acc.at[dv], add=True)
            return carry

        lax.fori_loop(0, nit, body, 0)
        plsc.subcore_barrier()
        for j in range(cpt):
            rb = rowbase + j * crow
            pltpu.sync_copy(acc.at[pl.ds(rb, crow)], obuf)
            pltpu.sync_copy(obuf, out_hbm.at[c, pl.ds(rb, crow)])

    return k(et, dst, oh, zeros_nd)


def _edge5_pass(xtab, src, dst, et, zeros_nd):
    E = src.shape[0]
    N = zeros_nd.shape[0]
    NW = _NC * _NS
    epw = E // NW
    nit = epw // _CHK
    rpt = N // _NS
    crow = 128
    cpt = rpt // crow
    dumpbase = N - 2048

    @functools.partial(
        pl.kernel,
        out_type=jax.ShapeDtypeStruct((_R, _NC, N, _D), jnp.float32),
        mesh=_mesh(),
        scratch_types=[
            pltpu.VMEM_SHARED((N, _D), jnp.float32),
            pltpu.VMEM((_CHK,), jnp.int32),
            pltpu.VMEM((_CHK,), jnp.int32),
            pltpu.VMEM((_CHK,), jnp.int32),
            pltpu.VMEM((_CHK,), jnp.int32),
            pltpu.VMEM((_CHK, _D), jnp.float32),
            pltpu.VMEM((crow, _D), jnp.float32),
            pltpu.SemaphoreType.DMA,
        ],
    )
    def k(x_hbm, src_hbm, dst_hbm, et_hbm, z_hbm, out_hbm,
          acc, sv, dv, tv, mv, rows, obuf, sem):
        c = lax.axis_index("c")
        s = lax.axis_index("s")
        wid = c * _NS + s
        rowbase = s * rpt
        ebase = wid * epw

        for r in range(_R):
            pltpu.sync_copy(z_hbm.at[pl.ds(rowbase, rpt)],
                            acc.at[pl.ds(rowbase, rpt)])
            plsc.subcore_barrier()

            def body(i, carry):
                off = ebase + i * _CHK
                pltpu.sync_copy(src_hbm.at[pl.ds(off, _CHK)], sv)
                pltpu.sync_copy(dst_hbm.at[pl.ds(off, _CHK)], dv)
                pltpu.sync_copy(et_hbm.at[pl.ds(off, _CHK)], tv)
                for j in range(_CHK // _L):
                    sl = pl.ds(j * _L, _L)
                    dump = (dumpbase + s * 128
                            + lax.iota(jnp.int32, _L) * 8
                            + lax.rem(i + j, 8))
                    mv[sl] = jnp.where(tv[sl] == r, dv[sl], dump)
                pltpu.async_copy(x_hbm.at[sv], rows, sem).wait()
                pltpu.sync_copy(rows, acc.at[mv], add=True)
                return carry

            lax.fori_loop(0, nit, body, 0)
            plsc.subcore_barrier()
            for j in range(cpt):
                rb = rowbase + j * crow
                pltpu.sync_copy(acc.at[pl.ds(rb, crow)], obuf)
                pltpu.sync_copy(obuf, out_hbm.at[r, c, pl.ds(rb, crow)])
            plsc.subcore_barrier()

    return k(xtab, src, dst, et, zeros_nd)


def _lrelu(v):
    return jnp.where(v >= 0, v, 0.01 * v)


def _front(des, tweet, num_prop, cat_prop,
           W_des, b_des, W_tw, b_tw, W_np, b_np, W_cp, b_cp, W_in, b_in):
    N = des.shape[0]
    B = 1000
    G = N // B

    def body(des_r, tw_r, np_r, cp_r,
             wdes_r, bdes_r, wtw_r, btw_r, wnp_r, bnp_r, wcp_r, bcp_r,
             win_r, bin_r, x_r):
        d = _lrelu(jnp.dot(des_r[...], wdes_r[...],
                           preferred_element_type=jnp.float32) + bdes_r[...])
        t = _lrelu(jnp.dot(tw_r[...], wtw_r[...],
                           preferred_element_type=jnp.float32) + btw_r[...])
        n = _lrelu(jnp.dot(np_r[...], wnp_r[...],
                           preferred_element_type=jnp.float32) + bnp_r[...])
        cc = _lrelu(jnp.dot(cp_r[...], wcp_r[...],
                            preferred_element_type=jnp.float32) + bcp_r[...])
        x = jnp.concatenate([d, t, n, cc], axis=1)
        x_r[...] = _lrelu(jnp.dot(x, win_r[...],
                                  preferred_element_type=jnp.float32) + bin_r[...])

    full = lambda a: pl.BlockSpec(a.shape, lambda i: (0,) * a.ndim)
    return pl.pallas_call(
        body,
        grid=(G,),
        in_specs=[
            pl.BlockSpec((B, 768), lambda i: (i, 0)),
            pl.BlockSpec((B, 768), lambda i: (i, 0)),
            pl.BlockSpec((B, 6), lambda i: (i, 0)),
            pl.BlockSpec((B, 11), lambda i: (i, 0)),
            full(W_des), full(b_des), full(W_tw), full(b_tw),
            full(W_np), full(b_np), full(W_cp), full(b_cp),
            full(W_in), full(b_in),
        ],
        out_specs=pl.BlockSpec((B, _D), lambda i: (i, 0)),
        out_shape=jax.ShapeDtypeStruct((N, _D), jnp.float32),
    )(des, tweet, num_prop, cat_prop,
      W_des, b_des, W_tw, b_tw, W_np, b_np, W_cp, b_cp, W_in, b_in)


def _rgcn_combine(body_tail, x, parts, cntp, consts, out_dim):
    N = x.shape[0]
    B = 1000
    G = N // B

    def body(x_r, p_r, cnt_r, wrel_r, wroot_r, brg_r, *rest):
        const_refs, out_r = rest[:-1], rest[-1]
        h = jnp.dot(x_r[...], wroot_r[...],
                    preferred_element_type=jnp.float32) + brg_r[...]
        cnt = cnt_r[0] + cnt_r[1]
        for r in range(_R):
            a = p_r[r, 0] + p_r[r, 1]
            cclip = jnp.maximum(cnt[:, r:r + 1], 1.0)
            h = h + jnp.dot(a / cclip, wrel_r[r],
                            preferred_element_type=jnp.float32)
        out_r[...] = body_tail(h, *[cr[...] for cr in const_refs])

    full = lambda a: pl.BlockSpec(a.shape, lambda i: (0,) * a.ndim)
    W_rel, W_root, b_rgcn = consts[:3]
    extra = consts[3:]
    return pl.pallas_call(
        body,
        grid=(G,),
        in_specs=[
            pl.BlockSpec((B, _D), lambda i: (i, 0)),
            pl.BlockSpec((_R, _NC, B, _D), lambda i: (0, 0, i, 0)),
            pl.BlockSpec((_NC, B, _D), lambda i: (0, i, 0)),
            full(W_rel), full(W_root), full(b_rgcn),
            *[full(a) for a in extra],
        ],
        out_specs=pl.BlockSpec((B, out_dim), lambda i: (i, 0)),
        out_shape=jax.ShapeDtypeStruct((N, out_dim), jnp.float32),
    )(x, parts, cntp, W_rel, W_root, b_rgcn, *extra)


def kernel(des, tweet, num_prop, cat_prop, edge_index, edge_type,
           W_des, b_des, W_tw, b_tw, W_np, b_np, W_cp, b_cp,
           W_in, b_in, W_rel, W_root, b_rgcn, W_o1, b_o1, W_o2, b_o2):
    N = des.shape[0]
    src = edge_index[0].astype(jnp.int32)
    dst = edge_index[1].astype(jnp.int32)
    et = edge_type.astype(jnp.int32)

    npad = ((N + 2048 + 2047) // 2048) * 2048
    oh = jnp.tile(jnp.eye(_R, _D, dtype=jnp.float32), (256, 1))
    zeros_nd = jnp.zeros((npad, _D), jnp.float32)
    r2 = lambda b: b.reshape(1, -1)

    cntp = _cnt_pass(et, dst, oh, zeros_nd)
    x1 = _front(des, tweet, num_prop, cat_prop,
                W_des, r2(b_des), W_tw, r2(b_tw), W_np, r2(b_np),
                W_cp, r2(b_cp), W_in, r2(b_in))
    p1 = _edge5_pass(x1, src, dst, et, zeros_nd)
    h1 = _rgcn_combine(lambda h: h, x1, p1, cntp,
                       (W_rel, W_root, r2(b_rgcn)), _D)
    p2 = _edge5_pass(h1, src, dst, et, zeros_nd)
    out = _rgcn_combine(
        lambda h, w1, b1, w2, b2: jnp.dot(
            _lrelu(jnp.dot(h, w1, preferred_element_type=jnp.float32) + b1),
            w2, preferred_element_type=jnp.float32) + b2,
        h1, p2, cntp,
        (W_rel, W_root, r2(b_rgcn), W_o1, r2(b_o1), W_o2, r2(b_o2)),
        2)
    return out


# speedup vs baseline: 2.0936x; 2.0936x over previous
import functools

import jax
import jax.numpy as jnp
from jax import lax
from jax.experimental import pallas as pl
from jax.experimental.pallas import tpu as pltpu
from jax.experimental.pallas import tpu_sc as plsc

_NC = 2
_NS = 16
_L = 16
_R = 5
_D = 128
_CHK = 80


def _mesh():
    return plsc.VectorSubcoreMesh(core_axis_name="c", subcore_axis_name="s")


def _cnt_pass(et, dst, oh, zeros_nd):
    E = et.shape[0]
    N = zeros_nd.shape[0]
    NW = _NC * _NS
    epw = E // NW
    nit = epw // _CHK
    rpt = N // _NS
    crow = 128
    cpt = rpt // crow

    @functools.partial(
        pl.kernel,
        out_type=jax.ShapeDtypeStruct((_NC, N, _D), jnp.float32),
        mesh=_mesh(),
        scratch_types=[
            pltpu.VMEM_SHARED((N, _D), jnp.float32),
            pltpu.VMEM((_CHK,), jnp.int32),
            pltpu.VMEM((_CHK,), jnp.int32),
            pltpu.VMEM((_CHK, _D), jnp.float32),
            pltpu.VMEM((crow, _D), jnp.float32),
            pltpu.SemaphoreType.DMA,
        ],
    )
    def k(et_hbm, dst_hbm, oh_hbm, z_hbm, out_hbm, acc, tv, dv, ohrows, obuf, sem):
        c = lax.axis_index("c")
        s = lax.axis_index("s")
        wid = s * _NC + c
        rowbase = s * rpt
        pltpu.sync_copy(z_hbm.at[pl.ds(rowbase, rpt)], acc.at[pl.ds(rowbase, rpt)])
        plsc.subcore_barrier()
        ebase = wid * epw
        nrep = oh_hbm.shape[0] // _R

        def body(i, carry):
            off = ebase + i * _CHK
            pltpu.sync_copy(et_hbm.at[pl.ds(off, _CHK)], tv)
            pltpu.sync_copy(dst_hbm.at[pl.ds(off, _CHK)], dv)
            rot = (i * (_CHK // _L) + wid) * 7
            for j in range(_CHK // _L):
                sl = pl.ds(j * _L, _L)
                m = lax.rem(lax.iota(jnp.int32, _L) * 3 + rot + j, nrep)
                tv[sl] = tv[sl] + m * _R
            pltpu.async_copy(oh_hbm.at[tv], ohrows, sem).wait()
            pltpu.sync_copy(ohrows, acc.at[dv], add=True)
            return carry

        lax.fori_loop(0, nit, body, 0)
        plsc.subcore_barrier()
        for j in range(cpt):
            rb = rowbase + j * crow
            pltpu.sync_copy(acc.at[pl.ds(rb, crow)], obuf)
            pltpu.sync_copy(obuf, out_hbm.at[c, pl.ds(rb, crow)])

    return k(et, dst, oh, zeros_nd)


def _edge5_pass(xtab, meta, zeros_nd):
    N = zeros_nd.shape[0]
    NW = _NC * _NS
    E = meta.shape[0] // 3
    epw = E // NW
    nit = epw // _CHK
    mw = 3 * _CHK
    rpt = N // _NS
    crow = 128
    cpt = rpt // crow
    dumpbase = N - 2048
    npairs = (nit + 1) // 2

    @functools.partial(
        pl.kernel,
        out_type=jax.ShapeDtypeStruct((_R, _NC, N, _D), jnp.float32),
        mesh=_mesh(),
        scratch_types=[
            pltpu.VMEM_SHARED((N, _D), jnp.float32),
            pltpu.VMEM((mw,), jnp.int32),
            pltpu.VMEM((mw,), jnp.int32),
            pltpu.VMEM((_CHK,), jnp.int32),
            pltpu.VMEM((_CHK,), jnp.int32),
            pltpu.VMEM((_CHK,), jnp.int32),
            pltpu.VMEM((_CHK, _D), jnp.float32),
            pltpu.VMEM((_CHK, _D), jnp.float32),
            pltpu.SemaphoreType.DMA,
            pltpu.SemaphoreType.DMA,
            pltpu.SemaphoreType.DMA,
            pltpu.SemaphoreType.DMA,
        ],
    )
    def k(x_hbm, meta_hbm, z_hbm, out_hbm,
          acc, m0, m1, g0, g1, mv, rows0, rows1, smg0, smg1, smm0, smm1):
        c = lax.axis_index("c")
        s = lax.axis_index("s")
        wid = c * _NS + s
        rowbase = s * rpt
        cbase = wid * nit
        bufs = ((m0, g0, rows0, smg0, smm0), (m1, g1, rows1, smg1, smm1))

        def meta_fetch(i, b):
            m = bufs[b][0]
            pltpu.async_copy(meta_hbm.at[pl.ds((cbase + i) * mw, mw)],
                             m, bufs[b][4])

        def meta_wait(i, b):
            m = bufs[b][0]
            pltpu.make_async_copy(
                meta_hbm.at[pl.ds((cbase + i) * mw, mw)], m, bufs[b][4]).wait()

        def gather(b):
            m, g, rows, smg, _ = bufs[b]
            for j in range(_CHK // _L):
                g[pl.ds(j * _L, _L)] = m[pl.ds(j * _L, _L)]
            pltpu.async_copy(x_hbm.at[g], rows, smg)

        def scatter(i, b, r):
            m, g, rows, smg, _ = bufs[b]
            pltpu.make_async_copy(x_hbm.at[g], rows, smg).wait()
            for j in range(_CHK // _L):
                dump = (dumpbase + s * 128
                        + lax.iota(jnp.int32, _L) * 8
                        + lax.rem(i + j, 8))
                mv[pl.ds(j * _L, _L)] = jnp.where(
                    m[pl.ds(2 * _CHK + j * _L, _L)] == r,
                    m[pl.ds(_CHK + j * _L, _L)], dump)
            pltpu.sync_copy(rows, acc.at[mv], add=True)

        def half(i, b, r):
            @pl.when(i + 1 < nit)
            def _():
                meta_wait(i + 1, 1 - b)
                gather(1 - b)
            scatter(i, b, r)
            @pl.when(i + 2 < nit)
            def _():
                meta_fetch(i + 2, b)

        for r in range(_R):
            pltpu.sync_copy(z_hbm.at[pl.ds(rowbase, rpt)],
                            acc.at[pl.ds(rowbase, rpt)])
            plsc.subcore_barrier()
            meta_fetch(0, 0)
            meta_wait(0, 0)
            gather(0)

            @pl.when(1 < nit)
            def _():
                meta_fetch(1, 1)

            def pair(gi, carry):
                i0 = 2 * gi
                half(i0, 0, r)

                @pl.when(i0 + 1 < nit)
                def _():
                    half(i0 + 1, 1, r)

                return carry

            lax.fori_loop(0, npairs, pair, 0)
            plsc.subcore_barrier()
            for j in range(cpt):
                rb = rowbase + j * crow
                pltpu.sync_copy(acc.at[pl.ds(rb, crow)],
                                out_hbm.at[r, c, pl.ds(rb, crow)])
            plsc.subcore_barrier()

    return k(xtab, meta, zeros_nd)


def _lrelu(v):
    return jnp.where(v >= 0, v, 0.01 * v)


def _front(des, tweet, num_prop, cat_prop,
           W_des, b_des, W_tw, b_tw, W_np, b_np, W_cp, b_cp, W_in, b_in):
    N = des.shape[0]
    B = 1000
    G = N // B

    def body(des_r, tw_r, np_r, cp_r,
             wdes_r, bdes_r, wtw_r, btw_r, wnp_r, bnp_r, wcp_r, bcp_r,
             win_r, bin_r, x_r):
        d = _lrelu(jnp.dot(des_r[...], wdes_r[...],
                           preferred_element_type=jnp.float32) + bdes_r[...])
        t = _lrelu(jnp.dot(tw_r[...], wtw_r[...],
                           preferred_element_type=jnp.float32) + btw_r[...])
        n = _lrelu(jnp.dot(np_r[...], wnp_r[...],
                           preferred_element_type=jnp.float32) + bnp_r[...])
        cc = _lrelu(jnp.dot(cp_r[...], wcp_r[...],
                            preferred_element_type=jnp.float32) + bcp_r[...])
        x = jnp.concatenate([d, t, n, cc], axis=1)
        x_r[...] = _lrelu(jnp.dot(x, win_r[...],
                                  preferred_element_type=jnp.float32) + bin_r[...])

    full = lambda a: pl.BlockSpec(a.shape, lambda i: (0,) * a.ndim)
    return pl.pallas_call(
        body,
        grid=(G,),
        in_specs=[
            pl.BlockSpec((B, 768), lambda i: (i, 0)),
            pl.BlockSpec((B, 768), lambda i: (i, 0)),
            pl.BlockSpec((B, 6), lambda i: (i, 0)),
            pl.BlockSpec((B, 11), lambda i: (i, 0)),
            full(W_des), full(b_des), full(W_tw), full(b_tw),
            full(W_np), full(b_np), full(W_cp), full(b_cp),
            full(W_in), full(b_in),
        ],
        out_specs=pl.BlockSpec((B, _D), lambda i: (i, 0)),
        out_shape=jax.ShapeDtypeStruct((N, _D), jnp.float32),
    )(des, tweet, num_prop, cat_prop,
      W_des, b_des, W_tw, b_tw, W_np, b_np, W_cp, b_cp, W_in, b_in)


def _rgcn_combine(body_tail, x, parts, cntp, consts, out_dim):
    N = x.shape[0]
    B = 1000
    G = N // B

    def body(x_r, p_r, cnt_r, wrel_r, wroot_r, brg_r, *rest):
        const_refs, out_r = rest[:-1], rest[-1]
        h = jnp.dot(x_r[...], wroot_r[...],
                    preferred_element_type=jnp.float32) + brg_r[...]
        cnt = cnt_r[0] + cnt_r[1]
        for r in range(_R):
            a = p_r[r, 0] + p_r[r, 1]
            cclip = jnp.maximum(cnt[:, r:r + 1], 1.0)
            h = h + jnp.dot(a / cclip, wrel_r[r],
                            preferred_element_type=jnp.float32)
        out_r[...] = body_tail(h, *[cr[...] for cr in const_refs])

    full = lambda a: pl.BlockSpec(a.shape, lambda i: (0,) * a.ndim)
    W_rel, W_root, b_rgcn = consts[:3]
    extra = consts[3:]
    return pl.pallas_call(
        body,
        grid=(G,),
        in_specs=[
            pl.BlockSpec((B, _D), lambda i: (i, 0)),
            pl.BlockSpec((_R, _NC, B, _D), lambda i: (0, 0, i, 0)),
            pl.BlockSpec((_NC, B, _D), lambda i: (0, i, 0)),
            full(W_rel), full(W_root), full(b_rgcn),
            *[full(a) for a in extra],
        ],
        out_specs=pl.BlockSpec((B, out_dim), lambda i: (i, 0)),
        out_shape=jax.ShapeDtypeStruct((N, out_dim), jnp.float32),
    )(x, parts, cntp, W_rel, W_root, b_rgcn, *extra)


def kernel(des, tweet, num_prop, cat_prop, edge_index, edge_type,
           W_des, b_des, W_tw, b_tw, W_np, b_np, W_cp, b_cp,
           W_in, b_in, W_rel, W_root, b_rgcn, W_o1, b_o1, W_o2, b_o2):
    N = des.shape[0]
    src = edge_index[0].astype(jnp.int32)
    dst = edge_index[1].astype(jnp.int32)
    et = edge_type.astype(jnp.int32)

    npad = ((N + 2048 + 2047) // 2048) * 2048
    oh = jnp.tile(jnp.eye(_R, _D, dtype=jnp.float32), (256, 1))
    zeros_nd = jnp.zeros((npad, _D), jnp.float32)
    r2 = lambda b: b.reshape(1, -1)

    E = src.shape[0]
    NW = _NC * _NS
    nit = E // NW // _CHK
    meta = (jnp.stack([src, dst, et], 0)
            .reshape(3, NW, nit, _CHK).transpose(1, 2, 0, 3).reshape(-1))

    cntp = _cnt_pass(et, dst, oh, zeros_nd)
    x1 = _front(des, tweet, num_prop, cat_prop,
                W_des, r2(b_des), W_tw, r2(b_tw), W_np, r2(b_np),
                W_cp, r2(b_cp), W_in, r2(b_in))
    p1 = _edge5_pass(x1, meta, zeros_nd)
    h1 = _rgcn_combine(lambda h: h, x1, p1, cntp,
                       (W_rel, W_root, r2(b_rgcn)), _D)
    p2 = _edge5_pass(h1, meta, zeros_nd)
    out = _rgcn_combine(
        lambda h, w1, b1, w2, b2: jnp.dot(
            _lrelu(jnp.dot(h, w1, preferred_element_type=jnp.float32) + b1),
            w2, preferred_element_type=jnp.float32) + b2,
        h1, p2, cntp,
        (W_rel, W_root, r2(b_rgcn), W_o1, r2(b_o1), W_o2, r2(b_o2)),
        2)
    return out
